# baseline (speedup 1.0000x reference)
import functools

import jax
import jax.numpy as jnp
from jax.experimental import pallas as pl
from jax.experimental.pallas import tpu as pltpu

_BB = 64
_NP = 88


def _tgcn_body(nt, x_ref, m_ref, a2_ref, c2_ref, uzr_ref, uh_ref, lw_ref,
               lb_ref, out_ref):
    n, t, hid = nt
    npad = x_ref.shape[1]
    bb = x_ref.shape[0]
    f = x_ref.shape[2] // t
    m = m_ref[...]
    x = x_ref[...]
    mx = jnp.concatenate(
        [jnp.dot(m, x[b], preferred_element_type=jnp.float32)
         for b in range(bb)], axis=0)
    a2 = a2_ref[...]
    c2 = c2_ref[...]
    uzr = uzr_ref[...]
    uh = uh_ref[...]
    hs = jnp.zeros((bb * npad, hid), dtype=jnp.float32)
    for ti in range(t):
        g = jnp.dot(mx[:, ti * f:(ti + 1) * f], a2,
                    preferred_element_type=jnp.float32) + c2
        zr = g[:, :2 * hid] + jnp.dot(hs, uzr,
                                      preferred_element_type=jnp.float32)
        z = jax.nn.sigmoid(zr[:, :hid])
        r = jax.nn.sigmoid(zr[:, hid:2 * hid])
        ht = jnp.tanh(g[:, 2 * hid:] + jnp.dot(
            hs * r, uh, preferred_element_type=jnp.float32))
        hs = z * hs + (1.0 - z) * ht
    hr = jax.nn.relu(hs).reshape(bb, npad, hid)
    mask = (jax.lax.broadcasted_iota(jnp.int32, (bb, npad, hid), 1)
            < n).astype(jnp.float32)
    pooled = jnp.sum(hr * mask, axis=1) * (1.0 / n)
    out_ref[...] = jnp.dot(pooled, lw_ref[...],
                           preferred_element_type=jnp.float32) + lb_ref[...]


def kernel(agent_obs, hideout_obs, timestep_obs, num_agents,
           last_k_fugitive_detections, edge_index,
           Wz, bz, Wr, br, Wh, bh,
           Lzw, Lzb, Lrw, Lrb, Lhw, Lhb, lin_w, lin_b):
    B, T, N, F = agent_obs.shape
    HID = Wz.shape[1]

    loop = jnp.arange(N)
    row = jnp.concatenate([edge_index[0].astype(jnp.int32), loop])
    col = jnp.concatenate([edge_index[1].astype(jnp.int32), loop])
    deg = jnp.zeros((N,), jnp.float32).at[col].add(1.0)
    dinv = jnp.where(deg > 0, 1.0 / jnp.sqrt(deg), 0.0)
    norm = dinv[row] * dinv[col]
    m = jnp.zeros((N, N), jnp.float32).at[col, row].add(norm)
    mp = jnp.zeros((_NP, _NP), jnp.float32).at[:N, :N].set(m)

    a2 = jnp.concatenate([Wz @ Lzw[:HID], Wr @ Lrw[:HID], Wh @ Lhw[:HID]],
                         axis=1)
    c2 = jnp.concatenate([bz @ Lzw[:HID] + Lzb, br @ Lrw[:HID] + Lrb,
                          bh @ Lhw[:HID] + Lhb])[None]
    uzr = jnp.concatenate([Lzw[HID:], Lrw[HID:]], axis=1)
    uh = Lhw[HID:]

    xnm = jnp.transpose(agent_obs, (0, 2, 1, 3)).reshape(B, N, T * F)
    xp = jnp.zeros((B, _NP, T * F), jnp.float32).at[:, :N].set(xnm)

    grid = (B // _BB,)
    pooled = pl.pallas_call(
        functools.partial(_tgcn_body, (N, T, HID)),
        grid=grid,
        in_specs=[
            pl.BlockSpec((_BB, _NP, T * F), lambda i: (i, 0, 0)),
            pl.BlockSpec((_NP, _NP), lambda i: (0, 0)),
            pl.BlockSpec((F, 3 * HID), lambda i: (0, 0)),
            pl.BlockSpec((1, 3 * HID), lambda i: (0, 0)),
            pl.BlockSpec((HID, 2 * HID), lambda i: (0, 0)),
            pl.BlockSpec((HID, HID), lambda i: (0, 0)),
            pl.BlockSpec((HID, lin_w.shape[1]), lambda i: (0, 0)),
            pl.BlockSpec((1, lin_w.shape[1]), lambda i: (0, 0)),
        ],
        out_specs=pl.BlockSpec((_BB, lin_w.shape[1]), lambda i: (i, 0)),
        out_shape=jax.ShapeDtypeStruct((B, lin_w.shape[1]), jnp.float32),
        compiler_params=pltpu.CompilerParams(
            dimension_semantics=("parallel",)),
    )(xp, mp, a2, c2, uzr, uh, lin_w, lin_b[None])

    return jnp.concatenate(
        [pooled, hideout_obs, timestep_obs, last_k_fugitive_detections],
        axis=-1)


# speedup vs baseline: 1.1363x; 1.1363x over previous
import functools

import jax
import jax.numpy as jnp
from jax.experimental import pallas as pl
from jax.experimental.pallas import tpu as pltpu

_BB = 16
_NP = 88


def _tgcn_body(nt, x_ref, m_ref, a2_ref, c2_ref, uzr_ref, uh_ref, lw_ref,
               lb_ref, out_ref):
    n, t, hid = nt
    npad = x_ref.shape[1]
    bb = x_ref.shape[0]
    f = x_ref.shape[2] // t
    m = m_ref[...]
    x = x_ref[...]
    mx = jnp.concatenate(
        [jnp.dot(m, x[b], preferred_element_type=jnp.float32)
         for b in range(bb)], axis=0)
    a2 = a2_ref[...]
    c2 = c2_ref[...]
    uzr = uzr_ref[...]
    uh = uh_ref[...]
    hs = jnp.zeros((bb * npad, hid), dtype=jnp.float32)
    for ti in range(t):
        g = jnp.dot(mx[:, ti * f:(ti + 1) * f], a2,
                    preferred_element_type=jnp.float32) + c2
        zr = g[:, :2 * hid] + jnp.dot(hs, uzr,
                                      preferred_element_type=jnp.float32)
        z = jax.nn.sigmoid(zr[:, :hid])
        r = jax.nn.sigmoid(zr[:, hid:2 * hid])
        ht = jnp.tanh(g[:, 2 * hid:] + jnp.dot(
            hs * r, uh, preferred_element_type=jnp.float32))
        hs = z * hs + (1.0 - z) * ht
    hr = jax.nn.relu(hs).reshape(bb, npad, hid)
    mask = (jax.lax.broadcasted_iota(jnp.int32, (bb, npad, hid), 1)
            < n).astype(jnp.float32)
    pooled = jnp.sum(hr * mask, axis=1) * (1.0 / n)
    out_ref[...] = jnp.dot(pooled, lw_ref[...],
                           preferred_element_type=jnp.float32) + lb_ref[...]


def kernel(agent_obs, hideout_obs, timestep_obs, num_agents,
           last_k_fugitive_detections, edge_index,
           Wz, bz, Wr, br, Wh, bh,
           Lzw, Lzb, Lrw, Lrb, Lhw, Lhb, lin_w, lin_b):
    B, T, N, F = agent_obs.shape
    HID = Wz.shape[1]

    loop = jnp.arange(N)
    row = jnp.concatenate([edge_index[0].astype(jnp.int32), loop])
    col = jnp.concatenate([edge_index[1].astype(jnp.int32), loop])
    deg = jnp.zeros((N,), jnp.float32).at[col].add(1.0)
    dinv = jnp.where(deg > 0, 1.0 / jnp.sqrt(deg), 0.0)
    norm = dinv[row] * dinv[col]
    m = jnp.zeros((N, N), jnp.float32).at[col, row].add(norm)
    mp = jnp.zeros((_NP, _NP), jnp.float32).at[:N, :N].set(m)

    a2 = jnp.concatenate([Wz @ Lzw[:HID], Wr @ Lrw[:HID], Wh @ Lhw[:HID]],
                         axis=1)
    c2 = jnp.concatenate([bz @ Lzw[:HID] + Lzb, br @ Lrw[:HID] + Lrb,
                          bh @ Lhw[:HID] + Lhb])[None]
    uzr = jnp.concatenate([Lzw[HID:], Lrw[HID:]], axis=1)
    uh = Lhw[HID:]

    xnm = jnp.transpose(agent_obs, (0, 2, 1, 3)).reshape(B, N, T * F)
    xp = jnp.zeros((B, _NP, T * F), jnp.float32).at[:, :N].set(xnm)

    grid = (B // _BB,)
    pooled = pl.pallas_call(
        functools.partial(_tgcn_body, (N, T, HID)),
        grid=grid,
        in_specs=[
            pl.BlockSpec((_BB, _NP, T * F), lambda i: (i, 0, 0)),
            pl.BlockSpec((_NP, _NP), lambda i: (0, 0)),
            pl.BlockSpec((F, 3 * HID), lambda i: (0, 0)),
            pl.BlockSpec((1, 3 * HID), lambda i: (0, 0)),
            pl.BlockSpec((HID, 2 * HID), lambda i: (0, 0)),
            pl.BlockSpec((HID, HID), lambda i: (0, 0)),
            pl.BlockSpec((HID, lin_w.shape[1]), lambda i: (0, 0)),
            pl.BlockSpec((1, lin_w.shape[1]), lambda i: (0, 0)),
        ],
        out_specs=pl.BlockSpec((_BB, lin_w.shape[1]), lambda i: (i, 0)),
        out_shape=jax.ShapeDtypeStruct((B, lin_w.shape[1]), jnp.float32),
        compiler_params=pltpu.CompilerParams(
            dimension_semantics=("parallel",)),
    )(xp, mp, a2, c2, uzr, uh, lin_w, lin_b[None])

    return jnp.concatenate(
        [pooled, hideout_obs, timestep_obs, last_k_fugitive_detections],
        axis=-1)
